# Initial kernel scaffold; baseline (speedup 1.0000x reference)
import jax
import jax.numpy as jnp
from jax.experimental import pallas as pl


def kernel(x, edge_index, adj_values, W):
    raise NotImplementedError("write your pallas kernel here")


# speedup vs baseline: 5.5825x; 5.5825x over previous
import functools

import jax
import jax.numpy as jnp
from jax import lax
from jax.experimental import pallas as pl
from jax.experimental.pallas import tpu as pltpu
from jax.experimental.pallas import tpu_sc as plsc

_NC = 2
_NS = 16
_L = 16
_CH = 128


def _sc_aggregate(x, src, dst, adj):
    N, D = x.shape
    E = src.shape[0]
    NW = _NC * _NS
    eb = E // NW
    nfull = eb // _CH
    tail = eb - nfull * _CH
    rpt = (N // (_NS * 8)) * 8
    left = N - _NS * rpt
    zfull = rpt // _CH
    zrem = rpt - zfull * _CH
    nj = D // _L

    mesh = plsc.VectorSubcoreMesh(core_axis_name="c", subcore_axis_name="s")

    scratch = [
        pltpu.VMEM((_CH,), jnp.int32),
        pltpu.VMEM((_CH,), jnp.int32),
        pltpu.VMEM((_CH,), jnp.float32),
        pltpu.VMEM((_CH, D), jnp.float32),
        pltpu.VMEM_SHARED((N, D), jnp.float32),
        pltpu.SemaphoreType.DMA,
    ]
    if tail:
        scratch += [
            pltpu.VMEM((tail,), jnp.int32),
            pltpu.VMEM((tail,), jnp.int32),
            pltpu.VMEM((tail,), jnp.float32),
            pltpu.VMEM((tail, D), jnp.float32),
        ]

    @functools.partial(
        pl.kernel,
        out_type=jax.ShapeDtypeStruct((_NC, N, D), jnp.float32),
        mesh=mesh,
        scratch_types=scratch,
    )
    def agg(x_hbm, src_hbm, dst_hbm, adj_hbm, out_hbm, *refs):
        if tail:
            srcv, dstv, adjv, msg, acc, sem, srct, dstt, adjt, msgt = refs
        else:
            srcv, dstv, adjv, msg, acc, sem = refs
        c = lax.axis_index("c")
        s = lax.axis_index("s")
        wid = c * _NS + s

        zero = jnp.zeros((_L,), jnp.float32)

        def zrow(r, carry):
            for j in range(nj):
                msg[r, pl.ds(j * _L, _L)] = zero
            return carry

        lax.fori_loop(0, _CH, zrow, 0)
        r0 = s * rpt
        for i in range(zfull):
            pltpu.sync_copy(msg, acc.at[pl.ds(r0 + i * _CH, _CH)])
        if zrem:
            pltpu.sync_copy(msg.at[pl.ds(0, zrem)],
                            acc.at[pl.ds(r0 + zfull * _CH, zrem)])
        if left:
            @pl.when(s == 0)
            def _():
                pltpu.sync_copy(msg.at[pl.ds(0, left)],
                                acc.at[pl.ds(_NS * rpt, left)])
        plsc.subcore_barrier()

        def scale_rows(msg_ref, adj_ref, n):
            def grp(g, carry):
                a16 = adj_ref[pl.ds(g * _L, _L)]
                for r in range(_L):
                    av = lax.broadcast(a16[r], (_L,))
                    row = g * _L + r
                    for j in range(nj):
                        sl = pl.ds(j * _L, _L)
                        msg_ref[row, sl] = msg_ref[row, sl] * av
                return carry

            lax.fori_loop(0, n // _L, grp, 0)

        e0 = wid * eb

        def chunk(k, carry):
            base = e0 + k * _CH
            pltpu.sync_copy(src_hbm.at[pl.ds(base, _CH)], srcv)
            pltpu.sync_copy(dst_hbm.at[pl.ds(base, _CH)], dstv)
            pltpu.sync_copy(adj_hbm.at[pl.ds(base, _CH)], adjv)
            pltpu.async_copy(x_hbm.at[srcv], msg, sem).wait()
            scale_rows(msg, adjv, _CH)
            pltpu.sync_copy(msg, acc.at[dstv], add=True)
            return carry

        lax.fori_loop(0, nfull, chunk, 0)

        if tail:
            base = e0 + nfull * _CH
            pltpu.sync_copy(src_hbm.at[pl.ds(base, tail)], srct)
            pltpu.sync_copy(dst_hbm.at[pl.ds(base, tail)], dstt)
            pltpu.sync_copy(adj_hbm.at[pl.ds(base, tail)], adjt)
            pltpu.async_copy(x_hbm.at[srct], msgt, sem).wait()
            scale_rows(msgt, adjt, tail)
            pltpu.sync_copy(msgt, acc.at[dstt], add=True)

        plsc.subcore_barrier()

        for i in range(zfull):
            sl = pl.ds(r0 + i * _CH, _CH)
            pltpu.sync_copy(acc.at[sl], out_hbm.at[c, sl])
        if zrem:
            sl = pl.ds(r0 + zfull * _CH, zrem)
            pltpu.sync_copy(acc.at[sl], out_hbm.at[c, sl])
        if left:
            @pl.when(s == 0)
            def _():
                sl = pl.ds(_NS * rpt, left)
                pltpu.sync_copy(acc.at[sl], out_hbm.at[c, sl])

    return agg(x, src, dst, adj)


def _tc_finish(partials, W):
    _, N, D = partials.shape
    blk = 1000

    def body(p_ref, w_ref, o_ref):
        acc = p_ref[0] + p_ref[1]
        h = jnp.dot(acc, w_ref[...], preferred_element_type=jnp.float32)
        o_ref[...] = jnp.maximum(h, 0.0)

    return pl.pallas_call(
        body,
        grid=(N // blk,),
        in_specs=[
            pl.BlockSpec((2, blk, D), lambda i: (0, i, 0)),
            pl.BlockSpec((D, D), lambda i: (0, 0)),
        ],
        out_specs=pl.BlockSpec((blk, D), lambda i: (i, 0)),
        out_shape=jax.ShapeDtypeStruct((N, D), jnp.float32),
    )(partials, W)


def kernel(x, edge_index, adj_values, W):
    src = edge_index[0]
    dst = edge_index[1]
    partials = _sc_aggregate(x, src, dst, adj_values)
    return _tc_finish(partials, W)
